# Initial kernel scaffold; baseline (speedup 1.0000x reference)
import jax
import jax.numpy as jnp
from jax.experimental import pallas as pl


def kernel(xq, xk, xv, Wq, Wk, Wv):
    raise NotImplementedError("write your pallas kernel here")


# speedup vs baseline: 234.5643x; 234.5643x over previous
import numpy as np
import jax
import jax.numpy as jnp
from jax.experimental import pallas as pl
from jax.experimental.pallas import tpu as pltpu

_GAMMA = 0.9865
_DELTA = _GAMMA * _GAMMA
_CHUNK = 256


def _retention_body(xq_ref, xqs_ref, xk_ref, xv_ref, wq_ref, wk_ref, wv_ref,
                    dmat_ref, qdec_ref, kdec_ref, out_ref, r_ref):
    c = pl.program_id(1)
    C = _CHUNK
    f32 = jnp.float32

    @pl.when(c == 0)
    def _():
        r_ref[...] = jnp.zeros_like(r_ref)

    wq = wq_ref[...]
    qs = jnp.dot(xqs_ref[0], wq, preferred_element_type=f32)
    k = jnp.dot(xk_ref[0], wk_ref[...], preferred_element_type=f32)
    v = jnp.dot(xv_ref[0], wv_ref[...], preferred_element_type=f32)

    wq1 = jnp.sum(wq, axis=1, keepdims=True)
    qsum = jnp.dot(xq_ref[0], wq1, preferred_element_type=f32)
    ksum = jnp.sum(k, axis=1, keepdims=True)
    qk = qsum * ksum

    a = jax.lax.dot_general(qs, k, (((1,), (1,)), ((), ())),
                            preferred_element_type=f32)
    rows = jax.lax.broadcasted_iota(jnp.int32, (C, C), 0)
    cols = jax.lax.broadcasted_iota(jnp.int32, (C, C), 1)
    a = a * dmat_ref[...] + jnp.where(rows == cols, qk, f32(0.0))

    r = r_ref[...]
    intra = jnp.dot(a, v, preferred_element_type=f32)
    inter = jnp.dot(qs * qdec_ref[...], r, preferred_element_type=f32)
    out_ref[0] = intra + inter

    ksc = k * kdec_ref[...]
    r_ref[...] = f32(_DELTA ** C) * r + jax.lax.dot_general(
        ksc, v, (((0,), (0,)), ((), ())), preferred_element_type=f32)


@jax.jit
def kernel(xq, xk, xv, Wq, Wk, Wv):
    B, S, D = xq.shape
    C = _CHUNK
    N = S // C

    i = np.arange(C)
    dmat = np.where(i[:, None] >= i[None, :],
                    _DELTA ** (i[:, None] - i[None, :] + 1.0),
                    0.0).astype(np.float32)
    qdec = (_DELTA ** (i + 2.0)).astype(np.float32).reshape(C, 1)
    kdec = (_DELTA ** (C - 1.0 - i)).astype(np.float32).reshape(C, 1)

    xqs = jnp.concatenate(
        [xq[:, 1:], jnp.zeros((B, 1, D), xq.dtype)], axis=1)

    return pl.pallas_call(
        _retention_body,
        grid=(B, N),
        in_specs=[
            pl.BlockSpec((1, C, D), lambda b, c: (b, c, 0)),
            pl.BlockSpec((1, C, D), lambda b, c: (b, c, 0)),
            pl.BlockSpec((1, C, D), lambda b, c: (b, c, 0)),
            pl.BlockSpec((1, C, D), lambda b, c: (b, c, 0)),
            pl.BlockSpec((D, D), lambda b, c: (0, 0)),
            pl.BlockSpec((D, D), lambda b, c: (0, 0)),
            pl.BlockSpec((D, D), lambda b, c: (0, 0)),
            pl.BlockSpec((C, C), lambda b, c: (0, 0)),
            pl.BlockSpec((C, 1), lambda b, c: (0, 0)),
            pl.BlockSpec((C, 1), lambda b, c: (0, 0)),
        ],
        out_specs=pl.BlockSpec((1, C, D), lambda b, c: (b, c, 0)),
        out_shape=jax.ShapeDtypeStruct((B, S, D), jnp.float32),
        scratch_shapes=[pltpu.VMEM((D, D), jnp.float32)],
        compiler_params=pltpu.CompilerParams(
            dimension_semantics=("parallel", "arbitrary"),
            vmem_limit_bytes=96 * 1024 * 1024,
        ),
        name="chunkwise_retention",
    )(xq, xqs, xk, xv, Wq, Wk, Wv,
      jnp.asarray(dmat), jnp.asarray(qdec), jnp.asarray(kdec))
